# baseline (speedup 1.0000x reference)
import numpy as np
import jax
import jax.numpy as jnp
from jax.experimental import pallas as pl
from jax.experimental.pallas import tpu as pltpu


def _interp_weights(n: int, nn: int) -> np.ndarray:
    x_fix = np.arange(n, dtype=np.float64)
    x_var = np.linspace(0.0, float(n - 1), nn)
    x_repeat = np.tile(x_var[:, None], (len(x_fix),))
    distances = np.abs(x_repeat - x_fix)
    x_indices = np.searchsorted(x_fix, x_var)
    weights = np.zeros_like(distances)
    idx = np.arange(len(x_indices))
    weights[idx, x_indices] = distances[idx, x_indices - 1]
    weights[idx, x_indices - 1] = distances[idx, x_indices]
    weights /= np.sum(weights, axis=1)[:, None]
    return weights.astype(np.float32)


_N = 64
_NN = 128
_W_NP = _interp_weights(_N, _NN)
_CHUNK = 16


def _upsample_kernel(x_ref, w_ref, o_ref):
    n, nn = _N, _NN
    X = x_ref[0].astype(jnp.bfloat16)
    W = w_ref[...].astype(jnp.bfloat16)
    Wt = W.T

    def dot(a, b):
        return jax.lax.dot(a, b, preferred_element_type=jnp.float32)

    Xs = jnp.swapaxes(X, 1, 2)
    t0 = dot(W, Xs.reshape(n, n * n)).astype(jnp.bfloat16)
    t0 = t0.reshape(nn, n, n)
    for i in range(nn // _CHUNK):
        t = t0[i * _CHUNK:(i + 1) * _CHUNK]
        t = dot(t.reshape(_CHUNK * n, n), Wt).astype(jnp.bfloat16)
        t = jnp.swapaxes(t.reshape(_CHUNK, n, nn), 1, 2)
        o_ref[0, i * _CHUNK:(i + 1) * _CHUNK] = dot(
            t.reshape(_CHUNK * nn, n), Wt).reshape(_CHUNK, nn, nn)


def kernel(x):
    B, C, D, H, Wd = x.shape
    xs = x.reshape(C, D, H, Wd)
    w = jnp.asarray(_W_NP)
    out = pl.pallas_call(
        _upsample_kernel,
        grid=(C,),
        in_specs=[
            pl.BlockSpec((1, D, H, Wd), lambda c: (c, 0, 0, 0)),
            pl.BlockSpec((_NN, _N), lambda c: (0, 0)),
        ],
        out_specs=pl.BlockSpec((1, _NN, _NN, _NN), lambda c: (c, 0, 0, 0)),
        out_shape=jax.ShapeDtypeStruct((C, _NN, _NN, _NN), jnp.float32),
        compiler_params=pltpu.CompilerParams(
            dimension_semantics=("parallel",)),
    )(xs, w)
    return out.reshape(B, C, _NN, _NN, _NN)


# speedup vs baseline: 1.0280x; 1.0280x over previous
import numpy as np
import jax
import jax.numpy as jnp
from jax.experimental import pallas as pl
from jax.experimental.pallas import tpu as pltpu


def _interp_weights(n: int, nn: int) -> np.ndarray:
    x_fix = np.arange(n, dtype=np.float64)
    x_var = np.linspace(0.0, float(n - 1), nn)
    x_repeat = np.tile(x_var[:, None], (len(x_fix),))
    distances = np.abs(x_repeat - x_fix)
    x_indices = np.searchsorted(x_fix, x_var)
    weights = np.zeros_like(distances)
    idx = np.arange(len(x_indices))
    weights[idx, x_indices] = distances[idx, x_indices - 1]
    weights[idx, x_indices - 1] = distances[idx, x_indices]
    weights /= np.sum(weights, axis=1)[:, None]
    return weights.astype(np.float32)


_N = 64
_NN = 128
_W_NP = _interp_weights(_N, _NN)
_CHUNK = 64


def _upsample_kernel(x_ref, w_ref, o_ref):
    n, nn = _N, _NN
    X = x_ref[0].astype(jnp.bfloat16)
    W = w_ref[...].astype(jnp.bfloat16)
    Wt = W.T

    def dot(a, b):
        return jax.lax.dot(a, b, preferred_element_type=jnp.float32)

    Xs = jnp.swapaxes(X, 1, 2)
    t0 = dot(W, Xs.reshape(n, n * n)).astype(jnp.bfloat16)
    t0 = t0.reshape(nn, n, n)
    for i in range(nn // _CHUNK):
        t = t0[i * _CHUNK:(i + 1) * _CHUNK]
        t = dot(t.reshape(_CHUNK * n, n), Wt).astype(jnp.bfloat16)
        t = jnp.swapaxes(t.reshape(_CHUNK, n, nn), 1, 2)
        o_ref[0, i * _CHUNK:(i + 1) * _CHUNK] = dot(
            t.reshape(_CHUNK * nn, n), Wt).reshape(_CHUNK, nn, nn)


def kernel(x):
    B, C, D, H, Wd = x.shape
    xs = x.reshape(C, D, H, Wd)
    w = jnp.asarray(_W_NP)
    out = pl.pallas_call(
        _upsample_kernel,
        grid=(C,),
        in_specs=[
            pl.BlockSpec((1, D, H, Wd), lambda c: (c, 0, 0, 0)),
            pl.BlockSpec((_NN, _N), lambda c: (0, 0)),
        ],
        out_specs=pl.BlockSpec((1, _NN, _NN, _NN), lambda c: (c, 0, 0, 0)),
        out_shape=jax.ShapeDtypeStruct((C, _NN, _NN, _NN), jnp.float32),
        compiler_params=pltpu.CompilerParams(
            dimension_semantics=("parallel",)),
    )(xs, w)
    return out.reshape(B, C, _NN, _NN, _NN)


# speedup vs baseline: 1.0317x; 1.0036x over previous
import numpy as np
import jax
import jax.numpy as jnp
from jax.experimental import pallas as pl
from jax.experimental.pallas import tpu as pltpu


def _interp_weights(n: int, nn: int) -> np.ndarray:
    x_fix = np.arange(n, dtype=np.float64)
    x_var = np.linspace(0.0, float(n - 1), nn)
    x_repeat = np.tile(x_var[:, None], (len(x_fix),))
    distances = np.abs(x_repeat - x_fix)
    x_indices = np.searchsorted(x_fix, x_var)
    weights = np.zeros_like(distances)
    idx = np.arange(len(x_indices))
    weights[idx, x_indices] = distances[idx, x_indices - 1]
    weights[idx, x_indices - 1] = distances[idx, x_indices]
    weights /= np.sum(weights, axis=1)[:, None]
    return weights.astype(np.float32)


_N = 64
_NN = 128
_W_NP = _interp_weights(_N, _NN)
_CHUNK = 32


def _upsample_kernel(x_ref, w_ref, o_ref):
    n, nn = _N, _NN
    X = x_ref[0].astype(jnp.bfloat16)
    W = w_ref[...].astype(jnp.bfloat16)
    Wt = W.T

    def dot(a, b):
        return jax.lax.dot(a, b, preferred_element_type=jnp.float32)

    Xs = jnp.swapaxes(X, 1, 2)
    t0 = dot(W, Xs.reshape(n, n * n)).astype(jnp.bfloat16)
    t0 = t0.reshape(nn, n, n)
    for i in range(nn // _CHUNK):
        t = t0[i * _CHUNK:(i + 1) * _CHUNK]
        t = dot(t.reshape(_CHUNK * n, n), Wt).astype(jnp.bfloat16)
        t = jnp.swapaxes(t.reshape(_CHUNK, n, nn), 1, 2)
        o_ref[0, i * _CHUNK:(i + 1) * _CHUNK] = dot(
            t.reshape(_CHUNK * nn, n), Wt).reshape(_CHUNK, nn, nn)


def kernel(x):
    B, C, D, H, Wd = x.shape
    xs = x.reshape(C, D, H, Wd)
    w = jnp.asarray(_W_NP)
    out = pl.pallas_call(
        _upsample_kernel,
        grid=(C,),
        in_specs=[
            pl.BlockSpec((1, D, H, Wd), lambda c: (c, 0, 0, 0)),
            pl.BlockSpec((_NN, _N), lambda c: (0, 0)),
        ],
        out_specs=pl.BlockSpec((1, _NN, _NN, _NN), lambda c: (c, 0, 0, 0)),
        out_shape=jax.ShapeDtypeStruct((C, _NN, _NN, _NN), jnp.float32),
        compiler_params=pltpu.CompilerParams(
            dimension_semantics=("parallel",)),
    )(xs, w)
    return out.reshape(B, C, _NN, _NN, _NN)


# speedup vs baseline: 1.0911x; 1.0576x over previous
import numpy as np
import jax
import jax.numpy as jnp
from jax.experimental import pallas as pl
from jax.experimental.pallas import tpu as pltpu


def _interp_weights(n: int, nn: int) -> np.ndarray:
    x_fix = np.arange(n, dtype=np.float64)
    x_var = np.linspace(0.0, float(n - 1), nn)
    x_repeat = np.tile(x_var[:, None], (len(x_fix),))
    distances = np.abs(x_repeat - x_fix)
    x_indices = np.searchsorted(x_fix, x_var)
    weights = np.zeros_like(distances)
    idx = np.arange(len(x_indices))
    weights[idx, x_indices] = distances[idx, x_indices - 1]
    weights[idx, x_indices - 1] = distances[idx, x_indices]
    weights /= np.sum(weights, axis=1)[:, None]
    return weights.astype(np.float32)


_N = 64
_NN = 128
_W_NP = _interp_weights(_N, _NN)
_CHUNK = 32


def _upsample_kernel(x_ref, w_ref, o_ref):
    n, nn = _N, _NN
    X = x_ref[0].astype(jnp.bfloat16)
    W = w_ref[...].astype(jnp.bfloat16)
    Wt = W.T

    def dot(a, b):
        return jax.lax.dot(a, b, preferred_element_type=jnp.float32)

    Xs = jnp.swapaxes(X, 1, 2)
    g0 = Xs[0:n - 1].astype(jnp.float32)
    g1 = Xs[1:n].astype(jnp.float32)

    def bfr(v):
        return v.astype(jnp.bfloat16).astype(jnp.float32)

    kv = jax.lax.broadcasted_iota(
        jnp.int32, (n - 1, 1, 1), 0).astype(jnp.float32)
    wa_odd = bfr((kv + 64.0) * (1.0 / 127.0))
    wa_evn = bfr((kv + 1.0) * (1.0 / 127.0))
    ga = (wa_odd * g0 + (1.0 - wa_odd) * g1).astype(jnp.bfloat16)
    gb = (wa_evn * g0 + (1.0 - wa_evn) * g1).astype(jnp.bfloat16)
    mid = jnp.stack([ga, gb], axis=1).reshape(2 * (n - 1), n, n)
    t0 = jnp.concatenate([Xs[0:1], mid, Xs[n - 1:n]], axis=0)
    for i in range(nn // _CHUNK):
        t = t0[i * _CHUNK:(i + 1) * _CHUNK]
        t = dot(t.reshape(_CHUNK * n, n), Wt).astype(jnp.bfloat16)
        t = jnp.swapaxes(t.reshape(_CHUNK, n, nn), 1, 2)
        o_ref[0, i * _CHUNK:(i + 1) * _CHUNK] = dot(
            t.reshape(_CHUNK * nn, n), Wt).reshape(_CHUNK, nn, nn)


def kernel(x):
    B, C, D, H, Wd = x.shape
    xs = x.reshape(C, D, H, Wd)
    w = jnp.asarray(_W_NP)
    out = pl.pallas_call(
        _upsample_kernel,
        grid=(C,),
        in_specs=[
            pl.BlockSpec((1, D, H, Wd), lambda c: (c, 0, 0, 0)),
            pl.BlockSpec((_NN, _N), lambda c: (0, 0)),
        ],
        out_specs=pl.BlockSpec((1, _NN, _NN, _NN), lambda c: (c, 0, 0, 0)),
        out_shape=jax.ShapeDtypeStruct((C, _NN, _NN, _NN), jnp.float32),
        compiler_params=pltpu.CompilerParams(
            dimension_semantics=("parallel",)),
    )(xs, w)
    return out.reshape(B, C, _NN, _NN, _NN)
